# baseline (speedup 1.0000x reference)
import jax
import jax.numpy as jnp
from jax.experimental import pallas as pl
from jax.experimental.pallas import tpu as pltpu

_PACK = 16


def _a2c_packed(xs_ref, xp_ref, wfs_ref, wfp_ref, bfs_ref, bfp_ref,
                wpc_ref, bpc_ref, wims_ref, wimp_ref, bim_ref,
                wp2_ref, wc2_ref, wim2_ref, bp2_ref, bc2_ref, bim2_ref,
                pol_ref, crit_ref, im_ref):
    fs = jnp.maximum(
        jnp.dot(xs_ref[...], wfs_ref[...], preferred_element_type=jnp.float32)
        + bfs_ref[...], 0.0)
    fp = jnp.maximum(
        jnp.dot(xp_ref[...], wfp_ref[...], preferred_element_type=jnp.float32)
        + bfp_ref[...], 0.0)

    for s in range(8):
        f2 = fs[:, 128 * s:128 * (s + 1)]
        f2p = fp[:, 128 * s:128 * (s + 1)]
        hpc = jnp.maximum(
            jnp.dot(f2, wpc_ref[...], preferred_element_type=jnp.float32)
            + bpc_ref[...], 0.0)
        him = jnp.maximum(
            jnp.dot(f2, wims_ref[...], preferred_element_type=jnp.float32)
            + jnp.dot(f2p, wimp_ref[...], preferred_element_type=jnp.float32)
            + bim_ref[...], 0.0)
        pol_ref[:, 8 * s:8 * (s + 1)] = (
            jnp.dot(hpc, wp2_ref[...], preferred_element_type=jnp.float32)
            + bp2_ref[...])
        crit_ref[:, 2 * s:2 * (s + 1)] = (
            jnp.dot(hpc, wc2_ref[...], preferred_element_type=jnp.float32)
            + bc2_ref[...])
        im_ref[:, 8 * s:8 * (s + 1)] = (
            jnp.dot(him, wim2_ref[...], preferred_element_type=jnp.float32)
            + bim2_ref[...])


def kernel(state, state_prev, wfb, w1, w2, bias):
    B, D = state.shape
    F = 64
    H = 32
    no = 4
    P = B // _PACK

    wf_s = wfb[:D, :F]
    wf_p = wfb[D:, F:]
    wp1 = w1[:F, 0:H]
    wc1 = w1[:F, H:2 * H]
    wim1_s = w1[:F, 2 * H:3 * H]
    wim1_p = w1[F:, 2 * H:3 * H]
    wp2 = w2[0:H, 0:no]
    wc2 = w2[H:2 * H, no:no + 1]
    wim2 = w2[2 * H:3 * H, no + 1:2 * no + 1]
    bf_s = bias[:, 0:F]
    bf_p = bias[:, F:2 * F]
    bp1 = bias[:, 128:128 + H]
    bc1 = bias[:, 128 + H:128 + 2 * H]
    bim1 = bias[:, 128 + 2 * H:128 + 3 * H]
    bp2 = bias[:, 256:256 + no]
    bc2 = bias[:, 256 + no:256 + no + 1]
    bim2 = bias[:, 256 + no + 1:256 + 2 * no + 1]

    eye16 = jnp.eye(_PACK, dtype=jnp.float32)
    eye2 = jnp.eye(2, dtype=jnp.float32)
    wf16_s = jnp.kron(eye16, wf_s)
    wf16_p = jnp.kron(eye16, wf_p)
    bf16_s = jnp.tile(bf_s, (1, _PACK))
    bf16_p = jnp.tile(bf_p, (1, _PACK))
    wpc = jnp.kron(eye2, jnp.concatenate([wp1, wc1], axis=1))
    bpc = jnp.tile(jnp.concatenate([bp1, bc1], axis=1), (1, 2))
    wims = jnp.kron(eye2, wim1_s)
    wimp = jnp.kron(eye2, wim1_p)
    bim = jnp.tile(bim1, (1, 2))
    zH = jnp.zeros((H, no), jnp.float32)
    wp2b = jnp.kron(eye2, jnp.concatenate([wp2, zH], axis=0))
    zH1 = jnp.zeros((H, 1), jnp.float32)
    wc2b = jnp.kron(eye2, jnp.concatenate([zH1, wc2], axis=0))
    wim2b = jnp.kron(eye2, wim2)
    bp2b = jnp.tile(bp2, (1, 2))
    bc2b = jnp.tile(bc2, (1, 2))
    bim2b = jnp.tile(bim2, (1, 2))

    xs = state.reshape(P, _PACK * D)
    xp = state_prev.reshape(P, _PACK * D)

    tp = P
    for cand in (2048, 1024, 512, 256, 128, 64, 32, 16, 8):
        if P % cand == 0:
            tp = cand
            break

    def full(a):
        return pl.BlockSpec(a.shape, lambda i: (0, 0))

    outs = pl.pallas_call(
        _a2c_packed,
        out_shape=[
            jax.ShapeDtypeStruct((P, _PACK * no), jnp.float32),
            jax.ShapeDtypeStruct((P, _PACK), jnp.float32),
            jax.ShapeDtypeStruct((P, _PACK * no), jnp.float32),
        ],
        grid=(P // tp,),
        in_specs=[
            pl.BlockSpec((tp, _PACK * D), lambda i: (i, 0)),
            pl.BlockSpec((tp, _PACK * D), lambda i: (i, 0)),
            full(wf16_s), full(wf16_p), full(bf16_s), full(bf16_p),
            full(wpc), full(bpc), full(wims), full(wimp), full(bim),
            full(wp2b), full(wc2b), full(wim2b),
            full(bp2b), full(bc2b), full(bim2b),
        ],
        out_specs=[
            pl.BlockSpec((tp, _PACK * no), lambda i: (i, 0)),
            pl.BlockSpec((tp, _PACK), lambda i: (i, 0)),
            pl.BlockSpec((tp, _PACK * no), lambda i: (i, 0)),
        ],
        compiler_params=pltpu.CompilerParams(
            dimension_semantics=("parallel",)),
    )(xs, xp, wf16_s, wf16_p, bf16_s, bf16_p,
      wpc, bpc, wims, wimp, bim,
      wp2b, wc2b, wim2b, bp2b, bc2b, bim2b)

    policy = outs[0].reshape(B, no)
    critic = outs[1].reshape(B, 1)
    im = outs[2].reshape(B, no)
    return policy, critic, im


# speedup vs baseline: 9.2286x; 9.2286x over previous
import jax
import jax.numpy as jnp
from jax.experimental import pallas as pl
from jax.experimental.pallas import tpu as pltpu


def _a2c_t(xs_ref, xp_ref, wfs_ref, wfp_ref, wpc_ref, wims_ref, wimp_ref,
           wp2_ref, wc2_ref, wim2_ref, pol_ref, crit_ref, im_ref):
    n = xs_ref.shape[1]
    ones = jnp.ones((1, n), jnp.float32)

    def mm(w, x):
        return jnp.dot(w, x, preferred_element_type=jnp.float32)

    xs = jnp.concatenate([xs_ref[...], ones], axis=0)
    xp = jnp.concatenate([xp_ref[...], ones], axis=0)
    fs = jnp.maximum(mm(wfs_ref[...], xs), 0.0)
    fp = jnp.maximum(mm(wfp_ref[...], xp), 0.0)

    fs1 = jnp.concatenate([fs, ones], axis=0)
    hpc = jnp.maximum(mm(wpc_ref[...], fs1), 0.0)
    him = jnp.maximum(mm(wims_ref[...], fs1)
                      + mm(wimp_ref[...], fp), 0.0)

    pol_ref[...] = mm(wp2_ref[...],
                      jnp.concatenate([hpc[0:32], ones], axis=0))
    crit_ref[...] = mm(wc2_ref[...],
                       jnp.concatenate([hpc[32:64], ones], axis=0))
    im_ref[...] = mm(wim2_ref[...],
                     jnp.concatenate([him, ones], axis=0))


def kernel(state, state_prev, wfb, w1, w2, bias):
    B, D = state.shape
    F = 64
    H = 32
    no = 4

    col = lambda v: v.reshape(-1, 1)
    wfs = jnp.concatenate([wfb[:D, :F].T, col(bias[0, 0:F])], axis=1)
    wfp = jnp.concatenate([wfb[D:, F:].T, col(bias[0, F:2 * F])], axis=1)
    wpc = jnp.concatenate(
        [jnp.concatenate([w1[:F, 0:H].T, w1[:F, H:2 * H].T], axis=0),
         col(bias[0, 128:128 + 2 * H])], axis=1)
    wims = jnp.concatenate(
        [w1[:F, 2 * H:3 * H].T, col(bias[0, 128 + 2 * H:128 + 3 * H])],
        axis=1)
    wimp = w1[F:, 2 * H:3 * H].T
    wp2 = jnp.concatenate(
        [w2[0:H, 0:no].T, col(bias[0, 256:256 + no])], axis=1)
    wc2 = jnp.concatenate(
        [w2[H:2 * H, no:no + 1].T, col(bias[0, 256 + no:257 + no])],
        axis=1)
    wim2 = jnp.concatenate(
        [w2[2 * H:3 * H, no + 1:2 * no + 1].T,
         col(bias[0, 257 + no:257 + 2 * no])], axis=1)

    xs = state.T
    xp = state_prev.T

    tb = B
    for cand in (8192, 4096, 2048, 1024, 512, 256, 128):
        if B % cand == 0:
            tb = cand
            break

    def full(a):
        return pl.BlockSpec(a.shape, lambda i: (0, 0))

    outs = pl.pallas_call(
        _a2c_t,
        out_shape=[
            jax.ShapeDtypeStruct((no, B), jnp.float32),
            jax.ShapeDtypeStruct((1, B), jnp.float32),
            jax.ShapeDtypeStruct((no, B), jnp.float32),
        ],
        grid=(B // tb,),
        in_specs=[
            pl.BlockSpec((D, tb), lambda i: (0, i)),
            pl.BlockSpec((D, tb), lambda i: (0, i)),
            full(wfs), full(wfp), full(wpc), full(wims), full(wimp),
            full(wp2), full(wc2), full(wim2),
        ],
        out_specs=[
            pl.BlockSpec((no, tb), lambda i: (0, i)),
            pl.BlockSpec((1, tb), lambda i: (0, i)),
            pl.BlockSpec((no, tb), lambda i: (0, i)),
        ],
        compiler_params=pltpu.CompilerParams(
            dimension_semantics=("parallel",)),
    )(xs, xp, wfs, wfp, wpc, wims, wimp, wp2, wc2, wim2)

    return outs[0].T, outs[1].T, outs[2].T


# speedup vs baseline: 11.3940x; 1.2346x over previous
import jax
import jax.numpy as jnp
from jax.experimental import pallas as pl
from jax.experimental.pallas import tpu as pltpu

_DN = (((0,), (0,)), ((), ()))


def _mmT(w, x):
    return jax.lax.dot_general(w, x, _DN, preferred_element_type=jnp.float32)


def _a2c_t(xs_ref, xp_ref, wfb_ref, w1_ref, w2_ref, bias_ref,
           pol_ref, crit_ref, im_ref):
    n = xs_ref.shape[1]
    ones = jnp.ones((1, n), jnp.float32)
    wfb = wfb_ref[...]
    w1 = w1_ref[...]
    w2 = w2_ref[...]
    bias = bias_ref[...]

    xs = jnp.concatenate([xs_ref[...], ones], axis=0)
    xp = jnp.concatenate([xp_ref[...], ones], axis=0)
    wfs = jnp.concatenate([wfb[0:8, 0:64], bias[:, 0:64]], axis=0)
    wfp = jnp.concatenate([wfb[8:16, 64:128], bias[:, 64:128]], axis=0)
    fs = jnp.maximum(_mmT(wfs, xs), 0.0)
    fp = jnp.maximum(_mmT(wfp, xp), 0.0)

    fs1 = jnp.concatenate([fs, ones], axis=0)
    wpc = jnp.concatenate([w1[0:64, 0:64], bias[:, 128:192]], axis=0)
    hpc = jnp.maximum(_mmT(wpc, fs1), 0.0)
    wim_s = jnp.concatenate([w1[0:64, 64:96], bias[:, 192:224]], axis=0)
    him = jnp.maximum(_mmT(wim_s, fs1)
                      + _mmT(w1[64:128, 64:96], fp), 0.0)

    wp2 = jnp.concatenate([w2[0:32, 0:4], bias[:, 256:260]], axis=0)
    pol_ref[...] = _mmT(wp2, jnp.concatenate([hpc[0:32], ones], axis=0))
    wc2 = jnp.concatenate([w2[32:64, 4:5], bias[:, 260:261]], axis=0)
    crit_ref[...] = _mmT(wc2, jnp.concatenate([hpc[32:64], ones], axis=0))
    wim2 = jnp.concatenate([w2[64:96, 5:9], bias[:, 261:265]], axis=0)
    im_ref[...] = _mmT(wim2, jnp.concatenate([him, ones], axis=0))


def kernel(state, state_prev, wfb, w1, w2, bias):
    B, D = state.shape
    no = 4

    xs = state.T
    xp = state_prev.T

    tb = B
    for cand in (16384, 8192, 4096, 2048, 1024, 512, 256, 128):
        if B % cand == 0:
            tb = cand
            break

    def full(a):
        return pl.BlockSpec(a.shape, lambda i: (0, 0))

    outs = pl.pallas_call(
        _a2c_t,
        out_shape=[
            jax.ShapeDtypeStruct((no, B), jnp.float32),
            jax.ShapeDtypeStruct((1, B), jnp.float32),
            jax.ShapeDtypeStruct((no, B), jnp.float32),
        ],
        grid=(B // tb,),
        in_specs=[
            pl.BlockSpec((D, tb), lambda i: (0, i)),
            pl.BlockSpec((D, tb), lambda i: (0, i)),
            full(wfb), full(w1), full(w2), full(bias),
        ],
        out_specs=[
            pl.BlockSpec((no, tb), lambda i: (0, i)),
            pl.BlockSpec((1, tb), lambda i: (0, i)),
            pl.BlockSpec((no, tb), lambda i: (0, i)),
        ],
        compiler_params=pltpu.CompilerParams(
            dimension_semantics=("parallel",)),
    )(xs, xp, wfb, w1, w2, bias)

    return outs[0].T, outs[1].T, outs[2].T
